# baseline (speedup 1.0000x reference)
import functools
import math

import jax
import jax.numpy as jnp
import numpy as np
from jax import lax
from jax.experimental import pallas as pl
from jax.experimental.pallas import tpu as pltpu
from jax.experimental.pallas import tpu_sc as plsc

_SIZE = 100000
_SMOOTHING = 0.1
_PAD_ID = 3

_EPS = np.float32(_SMOOTHING / (_SIZE - 2))
_TGT_COEFF = float(_EPS - np.float32(1.0 - _SMOOTHING))
_ROW_CONST = float(
    (_SIZE - 2) * (_EPS * np.log(_EPS))
    + np.float32(1.0 - _SMOOTHING) * np.log(np.float32(1.0 - _SMOOTHING))
)

_BC = 2048
_NW = 32
_LANES = 16


def _sc_gather_kernel(t_hbm, x_hbm, out_hbm, t_v, idx_v, val_v, acc_v, sem):
    n_per_w = t_hbm.shape[0] // _NW
    wid = lax.axis_index("s") * 2 + lax.axis_index("c")
    base = wid * n_per_w
    pltpu.sync_copy(t_hbm.at[pl.ds(base, n_per_w)], t_v)
    for j in range(n_per_w // _LANES):
        rows = base + j * _LANES + lax.iota(jnp.int32, _LANES)
        idx_v[pl.ds(j * _LANES, _LANES)] = (
            rows * _SIZE + t_v[pl.ds(j * _LANES, _LANES)]
        )
    pltpu.async_copy(x_hbm.at[idx_v], val_v, sem).wait()
    acc = jnp.zeros((_LANES,), jnp.float32)
    for j in range(n_per_w // _LANES):
        t_j = t_v[pl.ds(j * _LANES, _LANES)]
        v_j = val_v[pl.ds(j * _LANES, _LANES)]
        acc = acc + jnp.where(
            t_j != _PAD_ID, jnp.float32(_TGT_COEFF) * v_j, jnp.float32(0.0)
        )
    acc_v[...] = acc
    pltpu.sync_copy(acc_v, out_hbm.at[wid])


def _sc_gather(t, x_flat):
    n = t.shape[0]
    run = pl.kernel(
        _sc_gather_kernel,
        out_type=jax.ShapeDtypeStruct((_NW, _LANES), jnp.float32),
        mesh=plsc.VectorSubcoreMesh(core_axis_name="c", subcore_axis_name="s"),
        scratch_types=[
            pltpu.VMEM((n // _NW,), jnp.int32),
            pltpu.VMEM((n // _NW,), jnp.int32),
            pltpu.VMEM((n // _NW,), jnp.float32),
            pltpu.VMEM((_LANES,), jnp.float32),
            pltpu.SemaphoreType.DMA,
        ],
    )
    return run(t, x_flat)


def _kl_tc_kernel(t_ref, sc_ref, x_ref, out_ref):
    j = pl.program_id(0)
    nblk = pl.num_programs(0)

    t = t_ref[:, :]
    x = x_ref[:, :]
    n, bc = x.shape
    row_ok = t != _PAD_ID
    wrow = jnp.where(row_ok, -_EPS, jnp.float32(0.0))

    @pl.when(j == 0)
    def _init():
        count = jnp.sum(row_ok.astype(jnp.float32))
        corr3 = _EPS * jnp.sum(
            jnp.where(row_ok, x[:, _PAD_ID : _PAD_ID + 1], jnp.float32(0.0))
        )
        sc_total = jnp.sum(sc_ref[:, :])
        out_ref[:, :] = (
            jnp.float32(_ROW_CONST) * count + corr3 + sc_total
        ).reshape(1, 1)

    def _weighted(xb):
        p = lax.dot_general(
            wrow.reshape(1, n),
            xb,
            (((1,), (0,)), ((), ())),
            preferred_element_type=jnp.float32,
        )
        return jnp.sum(p).reshape(1, 1)

    @pl.when(j < nblk - 1)
    def _full_block():
        out_ref[:, :] += _weighted(x)

    @pl.when(j == nblk - 1)
    def _tail_block():
        col = lax.broadcasted_iota(jnp.int32, (n, bc), 1) + j * bc
        xm = jnp.where(col < _SIZE, x, jnp.float32(0.0))
        out_ref[:, :] += _weighted(xm)


@jax.jit
def _run(x, t):
    n = x.shape[0]
    sc_part = _sc_gather(t[:, 0], x.reshape(-1))
    nblk = pl.cdiv(_SIZE, _BC)
    out = pl.pallas_call(
        _kl_tc_kernel,
        grid=(nblk,),
        in_specs=[
            pl.BlockSpec((n, 1), lambda j: (0, 0)),
            pl.BlockSpec((_NW, _LANES), lambda j: (0, 0)),
            pl.BlockSpec((n, _BC), lambda j: (0, j)),
        ],
        out_specs=pl.BlockSpec((1, 1), lambda j: (0, 0)),
        out_shape=jax.ShapeDtypeStruct((1, 1), jnp.float32),
    )(t, sc_part, x)
    return out[0, 0]


def kernel(x, target, nwords):
    x2 = x.reshape(-1, _SIZE)
    t = target.reshape(-1).astype(jnp.int32)[:, None]
    kl = _run(x2, t)
    return kl / nwords


# speedup vs baseline: 1.9790x; 1.9790x over previous
import functools
import math

import jax
import jax.numpy as jnp
import numpy as np
from jax import lax
from jax.experimental import pallas as pl
from jax.experimental.pallas import tpu as pltpu

_SIZE = 100000
_SMOOTHING = 0.1
_PAD_ID = 3

_EPS = np.float32(_SMOOTHING / (_SIZE - 2))
_TGT_COEFF = float(_EPS - np.float32(1.0 - _SMOOTHING))
_ROW_CONST = float(
    (_SIZE - 2) * (_EPS * np.log(_EPS))
    + np.float32(1.0 - _SMOOTHING) * np.log(np.float32(1.0 - _SMOOTHING))
)

_BC = 2048


def _kl_kernel(t_ref, x_ref, out_ref):
    j = pl.program_id(0)
    nblk = pl.num_programs(0)

    t = t_ref[:, :]
    x = x_ref[:, :]
    n, bc = x.shape
    row_ok = t != _PAD_ID
    wrow = jnp.where(row_ok, -_EPS, jnp.float32(0.0))

    @pl.when(j == 0)
    def _init():
        count = jnp.sum(row_ok.astype(jnp.float32))
        corr3 = _EPS * jnp.sum(
            jnp.where(row_ok, x[:, _PAD_ID : _PAD_ID + 1], jnp.float32(0.0))
        )
        out_ref[:, :] = (jnp.float32(_ROW_CONST) * count + corr3).reshape(1, 1)

    t_loc = t - j * bc
    lane = lax.broadcasted_iota(jnp.int32, (n, bc), 1)
    hit = (lane == t_loc) & row_ok
    gathered = jnp.sum(jnp.where(hit, x, jnp.float32(0.0)))

    def _rowsum(xb):
        p = lax.dot_general(
            wrow.reshape(1, n),
            xb,
            (((1,), (0,)), ((), ())),
            preferred_element_type=jnp.float32,
        )
        return jnp.sum(p)

    @pl.when(j < nblk - 1)
    def _full_block():
        out_ref[:, :] += (
            _rowsum(x) + jnp.float32(_TGT_COEFF) * gathered
        ).reshape(1, 1)

    @pl.when(j == nblk - 1)
    def _tail_block():
        xm = jnp.where(lane + j * bc < _SIZE, x, jnp.float32(0.0))
        out_ref[:, :] += (
            _rowsum(xm) + jnp.float32(_TGT_COEFF) * gathered
        ).reshape(1, 1)


@jax.jit
def _run(x, t):
    n = x.shape[0]
    nblk = pl.cdiv(_SIZE, _BC)
    out = pl.pallas_call(
        _kl_kernel,
        grid=(nblk,),
        in_specs=[
            pl.BlockSpec((n, 1), lambda j: (0, 0)),
            pl.BlockSpec((n, _BC), lambda j: (0, j)),
        ],
        out_specs=pl.BlockSpec((1, 1), lambda j: (0, 0)),
        out_shape=jax.ShapeDtypeStruct((1, 1), jnp.float32),
    )(t, x)
    return out[0, 0]


def kernel(x, target, nwords):
    x2 = x.reshape(-1, _SIZE)
    t = target.reshape(-1).astype(jnp.int32)[:, None]
    kl = _run(x2, t)
    return kl / nwords
